# baseline (device time: 205519 ns/iter reference)
import jax
import jax.numpy as jnp
from jax import lax
from jax.experimental import pallas as pl
from jax.experimental.pallas import tpu as pltpu

N_DEV = 16
SLOTS = 4


def kernel(x, router_W, route_idx, expert_W, shared_W):
    n_tok, d = x.shape
    n_exp_local, _, h = expert_W.shape
    n_exp_total = N_DEV * n_exp_local

    def body(x_ref, rw_ref, idx_ref, ew_ref, sw_ref, out_ref,
             comm_ref, send_sems, recv_sems):
        my = lax.axis_index("i")
        left = lax.rem(my + N_DEV - 1, N_DEV)
        right = lax.rem(my + 1, N_DEV)

        barrier_sem = pltpu.get_barrier_semaphore()
        for nbr in (left, right):
            pl.semaphore_signal(
                barrier_sem, inc=1,
                device_id=(nbr,), device_id_type=pl.DeviceIdType.MESH,
            )
        pl.semaphore_wait(barrier_sem, 2)

        xv = x_ref[...]

        scores = jnp.dot(xv, rw_ref[...], preferred_element_type=jnp.float32)
        m = jnp.max(scores, axis=-1, keepdims=True)
        p = jnp.exp(scores - m)
        p = p / jnp.sum(p, axis=-1, keepdims=True)
        idx = idx_ref[...]
        cols = lax.broadcasted_iota(jnp.int32, (n_tok, n_exp_total), 1)
        gate = jnp.sum(p * (cols == idx).astype(jnp.float32),
                       axis=-1, keepdims=True)

        acc = jnp.dot(xv, sw_ref[...], preferred_element_type=jnp.float32)

        for hop in range(N_DEV):
            slot = hop % SLOTS
            if hop < N_DEV - 1:
                rdma = pltpu.make_async_remote_copy(
                    src_ref=ew_ref if hop == 0 else comm_ref.at[slot],
                    dst_ref=comm_ref.at[(hop + 1) % SLOTS],
                    send_sem=send_sems.at[slot],
                    recv_sem=recv_sems.at[(hop + 1) % SLOTS],
                    device_id=(right,),
                    device_id_type=pl.DeviceIdType.MESH,
                )
                rdma.start()
            src_dev = lax.rem(my - hop + N_DEV, N_DEV)
            e_base = n_exp_local * src_dev
            for k in range(n_exp_local):
                w = ew_ref[k] if hop == 0 else comm_ref[slot, k]
                y = jnp.dot(xv, w, preferred_element_type=jnp.float32)
                sel = (idx == (e_base + k)).astype(jnp.float32)
                acc = acc + (gate * sel) * y
            if hop < N_DEV - 1:
                rdma.wait()

        out_ref[...] = acc

    return pl.pallas_call(
        body,
        out_shape=jax.ShapeDtypeStruct((n_tok, h), jnp.float32),
        in_specs=[pl.BlockSpec(memory_space=pltpu.VMEM)] * 5,
        out_specs=pl.BlockSpec(memory_space=pltpu.VMEM),
        scratch_shapes=[
            pltpu.VMEM((SLOTS, n_exp_local, d, h), jnp.float32),
            pltpu.SemaphoreType.DMA((SLOTS,)),
            pltpu.SemaphoreType.DMA((SLOTS,)),
        ],
        compiler_params=pltpu.CompilerParams(collective_id=0),
    )(x, router_W, route_idx, expert_W, shared_W)


# device time: 99221 ns/iter; 2.0713x vs baseline; 2.0713x over previous
import jax
import jax.numpy as jnp
from jax import lax
from jax.experimental import pallas as pl
from jax.experimental.pallas import tpu as pltpu

N_DEV = 16
SLOTS = 4


def kernel(x, router_W, route_idx, expert_W, shared_W):
    n_tok, d = x.shape
    n_exp_local, _, h = expert_W.shape
    n_exp_total = N_DEV * n_exp_local

    def body(x_ref, rw_ref, idx_ref, ew_ref, sw_ref, out_ref,
             stage_ref, cw_ref, ccw_ref,
             send_cw, recv_cw, send_ccw, recv_ccw):
        my = lax.axis_index("i")
        left = lax.rem(my + N_DEV - 1, N_DEV)
        right = lax.rem(my + 1, N_DEV)

        barrier_sem = pltpu.get_barrier_semaphore()
        for nbr in (left, right):
            pl.semaphore_signal(
                barrier_sem, inc=1,
                device_id=(nbr,), device_id_type=pl.DeviceIdType.MESH,
            )
        pl.semaphore_wait(barrier_sem, 2)

        stage_ref[0] = ew_ref[0].astype(jnp.bfloat16)
        stage_ref[1] = ew_ref[1].astype(jnp.bfloat16)

        xv = x_ref[...]
        xb = xv.astype(jnp.bfloat16)

        scores = jnp.dot(xv, rw_ref[...], preferred_element_type=jnp.float32)
        m = jnp.max(scores, axis=-1, keepdims=True)
        p = jnp.exp(scores - m)
        p = p / jnp.sum(p, axis=-1, keepdims=True)
        idx = idx_ref[...]
        cols = lax.broadcasted_iota(jnp.int32, (n_tok, n_exp_total), 1)
        gate = jnp.sum(p * (cols == idx).astype(jnp.float32),
                       axis=-1, keepdims=True)

        acc = jnp.dot(xv, sw_ref[...], preferred_element_type=jnp.float32)

        for hop in range(N_DEV):
            slot = hop % SLOTS
            nxt = (hop + 1) % SLOTS
            if hop < N_DEV - 1:
                rdma_cw = pltpu.make_async_remote_copy(
                    src_ref=stage_ref.at[0] if hop == 0 else cw_ref.at[slot],
                    dst_ref=cw_ref.at[nxt],
                    send_sem=send_cw.at[slot],
                    recv_sem=recv_cw.at[nxt],
                    device_id=(right,),
                    device_id_type=pl.DeviceIdType.MESH,
                )
                rdma_ccw = pltpu.make_async_remote_copy(
                    src_ref=stage_ref.at[1] if hop == 0 else ccw_ref.at[slot],
                    dst_ref=ccw_ref.at[nxt],
                    send_sem=send_ccw.at[slot],
                    recv_sem=recv_ccw.at[nxt],
                    device_id=(left,),
                    device_id_type=pl.DeviceIdType.MESH,
                )
                rdma_cw.start()
                rdma_ccw.start()

            e_cw = n_exp_local * lax.rem(my - hop + N_DEV, N_DEV)
            e_ccw = n_exp_local * lax.rem(my + hop, N_DEV) + 1
            w_cw = stage_ref[0] if hop == 0 else cw_ref[slot]
            w_ccw = stage_ref[1] if hop == 0 else ccw_ref[slot]
            for e_id, w in ((e_cw, w_cw), (e_ccw, w_ccw)):
                y = jnp.dot(xb, w, preferred_element_type=jnp.float32)
                sel = (idx == e_id).astype(jnp.float32)
                acc = acc + (gate * sel) * y
            if hop < N_DEV - 1:
                rdma_cw.wait()
                rdma_ccw.wait()

        out_ref[...] = acc

    return pl.pallas_call(
        body,
        out_shape=jax.ShapeDtypeStruct((n_tok, h), jnp.float32),
        in_specs=[pl.BlockSpec(memory_space=pltpu.VMEM)] * 5,
        out_specs=pl.BlockSpec(memory_space=pltpu.VMEM),
        scratch_shapes=[
            pltpu.VMEM((n_exp_local, d, h), jnp.bfloat16),
            pltpu.VMEM((SLOTS, d, h), jnp.bfloat16),
            pltpu.VMEM((SLOTS, d, h), jnp.bfloat16),
            pltpu.SemaphoreType.DMA((SLOTS,)),
            pltpu.SemaphoreType.DMA((SLOTS,)),
            pltpu.SemaphoreType.DMA((SLOTS,)),
            pltpu.SemaphoreType.DMA((SLOTS,)),
        ],
        compiler_params=pltpu.CompilerParams(collective_id=0),
    )(x, router_W, route_idx, expert_W, shared_W)


# device time: 68627 ns/iter; 2.9947x vs baseline; 1.4458x over previous
import jax
import jax.numpy as jnp
from jax import lax
from jax.experimental import pallas as pl
from jax.experimental.pallas import tpu as pltpu

N_DEV = 16
SLOTS = 4
C = 2


def kernel(x, router_W, route_idx, expert_W, shared_W):
    n_tok, d = x.shape
    n_exp_local, _, h = expert_W.shape
    n_exp_total = N_DEV * n_exp_local
    dc = d // C

    def body(x_ref, rw_ref, idx_ref, ew_ref, sw_ref, out_ref,
             stage_ref, cw_ref, ccw_ref,
             send_cw, recv_cw, send_ccw, recv_ccw):
        my = lax.axis_index("i")
        left = lax.rem(my + N_DEV - 1, N_DEV)
        right = lax.rem(my + 1, N_DEV)

        barrier_sem = pltpu.get_barrier_semaphore()
        for nbr in (left, right):
            pl.semaphore_signal(
                barrier_sem, inc=1,
                device_id=(nbr,), device_id_type=pl.DeviceIdType.MESH,
            )
        pl.semaphore_wait(barrier_sem, 2)

        stage_ref[0] = ew_ref[0].astype(jnp.bfloat16)
        stage_ref[1] = ew_ref[1].astype(jnp.bfloat16)

        def fwd(dirn, hop, c):
            buf, s_sems, r_sems, st, nbr = (
                (cw_ref, send_cw, recv_cw, 0, right) if dirn == 0
                else (ccw_ref, send_ccw, recv_ccw, 1, left)
            )
            slot, nxt = hop % SLOTS, (hop + 1) % SLOTS
            rows = pl.ds(c * dc, dc)
            return pltpu.make_async_remote_copy(
                src_ref=(stage_ref.at[st, rows, :] if hop == 0
                         else buf.at[slot, rows, :]),
                dst_ref=buf.at[nxt, rows, :],
                send_sem=s_sems.at[slot, c],
                recv_sem=r_sems.at[nxt, c],
                device_id=(nbr,),
                device_id_type=pl.DeviceIdType.MESH,
            )

        last_f = {}
        for c in range(C):
            for dirn in (0, 1):
                f = fwd(dirn, 0, c)
                f.start()
                last_f[(dirn, c)] = f

        xv = x_ref[...]
        xb = xv.astype(jnp.bfloat16)

        scores = jnp.dot(xv, rw_ref[...], preferred_element_type=jnp.float32)
        m = jnp.max(scores, axis=-1, keepdims=True)
        p = jnp.exp(scores - m)
        p = p / jnp.sum(p, axis=-1, keepdims=True)
        idx = idx_ref[...]
        cols = lax.broadcasted_iota(jnp.int32, (n_tok, n_exp_total), 1)
        gate = jnp.sum(p * (cols == idx).astype(jnp.float32),
                       axis=-1, keepdims=True)

        acc = jnp.dot(xv, sw_ref[...], preferred_element_type=jnp.float32)

        for hop in range(N_DEV):
            slot = hop % SLOTS
            if hop > 0:
                new_f = {}
                for c in range(C):
                    for dirn in (0, 1):
                        last_f[(dirn, c)].wait_recv()
                        if hop < N_DEV - 1:
                            f = fwd(dirn, hop, c)
                            f.start()
                            new_f[(dirn, c)] = f
                for f in last_f.values():
                    f.wait_send()
                last_f = new_f

            e_cw = n_exp_local * lax.rem(my - hop + N_DEV, N_DEV)
            e_ccw = n_exp_local * lax.rem(my + hop, N_DEV) + 1
            w_cw = stage_ref[0] if hop == 0 else cw_ref[slot]
            w_ccw = stage_ref[1] if hop == 0 else ccw_ref[slot]
            for e_id, w in ((e_cw, w_cw), (e_ccw, w_ccw)):
                y = jnp.dot(xb, w, preferred_element_type=jnp.float32)
                sel = (idx == e_id).astype(jnp.float32)
                acc = acc + (gate * sel) * y

        out_ref[...] = acc

    return pl.pallas_call(
        body,
        out_shape=jax.ShapeDtypeStruct((n_tok, h), jnp.float32),
        in_specs=[pl.BlockSpec(memory_space=pltpu.VMEM)] * 5,
        out_specs=pl.BlockSpec(memory_space=pltpu.VMEM),
        scratch_shapes=[
            pltpu.VMEM((n_exp_local, d, h), jnp.bfloat16),
            pltpu.VMEM((SLOTS, d, h), jnp.bfloat16),
            pltpu.VMEM((SLOTS, d, h), jnp.bfloat16),
            pltpu.SemaphoreType.DMA((SLOTS, C)),
            pltpu.SemaphoreType.DMA((SLOTS, C)),
            pltpu.SemaphoreType.DMA((SLOTS, C)),
            pltpu.SemaphoreType.DMA((SLOTS, C)),
        ],
        compiler_params=pltpu.CompilerParams(collective_id=0),
    )(x, router_W, route_idx, expert_W, shared_W)


# device time: 65509 ns/iter; 3.1373x vs baseline; 1.0476x over previous
import jax
import jax.numpy as jnp
from jax import lax
from jax.experimental import pallas as pl
from jax.experimental.pallas import tpu as pltpu

N_DEV = 16
SLOTS = 4
C = 4


def kernel(x, router_W, route_idx, expert_W, shared_W):
    n_tok, d = x.shape
    n_exp_local, _, h = expert_W.shape
    n_exp_total = N_DEV * n_exp_local
    dc = d // C

    def body(x_ref, rw_ref, idx_ref, ew_ref, sw_ref, out_ref,
             stage_ref, cw_ref, ccw_ref,
             send_cw, recv_cw, send_ccw, recv_ccw):
        my = lax.axis_index("i")
        left = lax.rem(my + N_DEV - 1, N_DEV)
        right = lax.rem(my + 1, N_DEV)

        barrier_sem = pltpu.get_barrier_semaphore()
        for nbr in (left, right):
            pl.semaphore_signal(
                barrier_sem, inc=1,
                device_id=(nbr,), device_id_type=pl.DeviceIdType.MESH,
            )
        pl.semaphore_wait(barrier_sem, 2)

        stage_ref[0] = ew_ref[0].astype(jnp.bfloat16)
        stage_ref[1] = ew_ref[1].astype(jnp.bfloat16)

        def fwd(dirn, hop, c):
            buf, s_sems, r_sems, st, nbr = (
                (cw_ref, send_cw, recv_cw, 0, right) if dirn == 0
                else (ccw_ref, send_ccw, recv_ccw, 1, left)
            )
            slot, nxt = hop % SLOTS, (hop + 1) % SLOTS
            rows = pl.ds(c * dc, dc)
            return pltpu.make_async_remote_copy(
                src_ref=(stage_ref.at[st, rows, :] if hop == 0
                         else buf.at[slot, rows, :]),
                dst_ref=buf.at[nxt, rows, :],
                send_sem=s_sems.at[slot, c],
                recv_sem=r_sems.at[nxt, c],
                device_id=(nbr,),
                device_id_type=pl.DeviceIdType.MESH,
            )

        last_f = {}
        for c in range(C):
            for dirn in (0, 1):
                f = fwd(dirn, 0, c)
                f.start()
                last_f[(dirn, c)] = f

        xv = x_ref[...]
        xb = xv.astype(jnp.bfloat16)

        scores = jnp.dot(xv, rw_ref[...], preferred_element_type=jnp.float32)
        m = jnp.max(scores, axis=-1, keepdims=True)
        p = jnp.exp(scores - m)
        p = p / jnp.sum(p, axis=-1, keepdims=True)
        idx = idx_ref[...]
        cols = lax.broadcasted_iota(jnp.int32, (n_tok, n_exp_total), 1)
        gate = jnp.sum(p * (cols == idx).astype(jnp.float32),
                       axis=-1, keepdims=True)

        acc = jnp.dot(xv, sw_ref[...], preferred_element_type=jnp.float32)

        for hop in range(N_DEV):
            slot = hop % SLOTS
            if hop > 0:
                new_f = {}
                for c in range(C):
                    for dirn in (0, 1):
                        last_f[(dirn, c)].wait_recv()
                        if hop < N_DEV - 1:
                            f = fwd(dirn, hop, c)
                            f.start()
                            new_f[(dirn, c)] = f
                for f in last_f.values():
                    f.wait_send()
                last_f = new_f

            e_cw = n_exp_local * lax.rem(my - hop + N_DEV, N_DEV)
            e_ccw = n_exp_local * lax.rem(my + hop, N_DEV) + 1
            w_cw = stage_ref[0] if hop == 0 else cw_ref[slot]
            w_ccw = stage_ref[1] if hop == 0 else ccw_ref[slot]
            for e_id, w in ((e_cw, w_cw), (e_ccw, w_ccw)):
                y = jnp.dot(xb, w, preferred_element_type=jnp.float32)
                sel = (idx == e_id).astype(jnp.float32)
                acc = acc + (gate * sel) * y

        out_ref[...] = acc

    return pl.pallas_call(
        body,
        out_shape=jax.ShapeDtypeStruct((n_tok, h), jnp.float32),
        in_specs=[pl.BlockSpec(memory_space=pltpu.VMEM)] * 5,
        out_specs=pl.BlockSpec(memory_space=pltpu.VMEM),
        scratch_shapes=[
            pltpu.VMEM((n_exp_local, d, h), jnp.bfloat16),
            pltpu.VMEM((SLOTS, d, h), jnp.bfloat16),
            pltpu.VMEM((SLOTS, d, h), jnp.bfloat16),
            pltpu.SemaphoreType.DMA((SLOTS, C)),
            pltpu.SemaphoreType.DMA((SLOTS, C)),
            pltpu.SemaphoreType.DMA((SLOTS, C)),
            pltpu.SemaphoreType.DMA((SLOTS, C)),
        ],
        compiler_params=pltpu.CompilerParams(collective_id=0),
    )(x, router_W, route_idx, expert_W, shared_W)


# device time: 58919 ns/iter; 3.4882x vs baseline; 1.1118x over previous
import jax
import jax.numpy as jnp
from jax import lax
from jax.experimental import pallas as pl
from jax.experimental.pallas import tpu as pltpu

N_DEV = 16
SLOTS = 4
C = 2

CW, CCW = 0, 1
SEND_HOPS = {CW: range(0, 8), CCW: range(0, 7)}
LAST_HOP = 8


def kernel(x, router_W, route_idx, expert_W, shared_W):
    n_tok, d = x.shape
    n_exp_local, _, h = expert_W.shape
    n_exp_total = N_DEV * n_exp_local
    assert C in (2, 4)
    if C == 2:
        chunk_k = (0, 1)
        chunk_rows = (pl.ds(0, d), pl.ds(0, d))
    else:
        chunk_k = (0, 0, 1, 1)
        chunk_rows = tuple(pl.ds((c % 2) * (d // 2), d // 2) for c in range(4))

    def body(x_ref, rw_ref, idx_ref, ew_ref, sw_ref, out_ref,
             stage_ref, cw_ref, ccw_ref,
             send_cw, recv_cw, send_ccw, recv_ccw):
        my = lax.axis_index("i")
        left = lax.rem(my + N_DEV - 1, N_DEV)
        right = lax.rem(my + 1, N_DEV)

        barrier_sem = pltpu.get_barrier_semaphore()
        for nbr in (left, right):
            pl.semaphore_signal(
                barrier_sem, inc=1,
                device_id=(nbr,), device_id_type=pl.DeviceIdType.MESH,
            )
        pl.semaphore_wait(barrier_sem, 2)

        stage_ref[0] = ew_ref[0].astype(jnp.bfloat16)
        stage_ref[1] = ew_ref[1].astype(jnp.bfloat16)

        def fwd(dirn, hop, c):
            buf, s_sems, r_sems, nbr = (
                (cw_ref, send_cw, recv_cw, right) if dirn == CW
                else (ccw_ref, send_ccw, recv_ccw, left)
            )
            slot, nxt = hop % SLOTS, (hop + 1) % SLOTS
            k, rows = chunk_k[c], chunk_rows[c]
            return pltpu.make_async_remote_copy(
                src_ref=(stage_ref.at[k, rows, :] if hop == 0
                         else buf.at[slot, k, rows, :]),
                dst_ref=buf.at[nxt, k, rows, :],
                send_sem=s_sems.at[slot, c],
                recv_sem=r_sems.at[nxt, c],
                device_id=(nbr,),
                device_id_type=pl.DeviceIdType.MESH,
            )

        last_f = {}
        for c in range(C):
            for dirn in (CW, CCW):
                f = fwd(dirn, 0, c)
                f.start()
                last_f[(dirn, c)] = f

        xv = x_ref[...]
        xb = xv.astype(jnp.bfloat16)

        scores = jnp.dot(xv, rw_ref[...], preferred_element_type=jnp.float32)
        m = jnp.max(scores, axis=-1, keepdims=True)
        p = jnp.exp(scores - m)
        p = p / jnp.sum(p, axis=-1, keepdims=True)
        idx = idx_ref[...]
        cols = lax.broadcasted_iota(jnp.int32, (n_tok, n_exp_total), 1)
        gate = jnp.sum(p * (cols == idx).astype(jnp.float32),
                       axis=-1, keepdims=True)

        acc = jnp.dot(xv, sw_ref[...], preferred_element_type=jnp.float32)

        def contribute(acc, e_base, w_pair_read):
            for k in range(n_exp_local):
                y = jnp.dot(xb, w_pair_read(k),
                            preferred_element_type=jnp.float32)
                sel = (idx == (e_base + k)).astype(jnp.float32)
                acc = acc + (gate * sel) * y
            return acc

        acc = contribute(acc, n_exp_local * my, lambda k: stage_ref[k])

        for hop in range(1, LAST_HOP + 1):
            slot = hop % SLOTS
            new_f = {}
            arrived = []
            for c in range(C):
                for dirn in (CW, CCW):
                    if (hop - 1) in SEND_HOPS[dirn]:
                        last_f[(dirn, c)].wait_recv()
                        arrived.append(last_f[(dirn, c)])
                        if hop in SEND_HOPS[dirn]:
                            f = fwd(dirn, hop, c)
                            f.start()
                            new_f[(dirn, c)] = f
            for f in arrived:
                f.wait_send()
            last_f = new_f

            e_cw = n_exp_local * lax.rem(my - hop + N_DEV, N_DEV)
            acc = contribute(acc, e_cw, lambda k: cw_ref[slot, k])
            if hop <= 7:
                e_ccw = n_exp_local * lax.rem(my + hop, N_DEV)
                acc = contribute(acc, e_ccw, lambda k: ccw_ref[slot, k])

        out_ref[...] = acc

    return pl.pallas_call(
        body,
        out_shape=jax.ShapeDtypeStruct((n_tok, h), jnp.float32),
        in_specs=[pl.BlockSpec(memory_space=pltpu.VMEM)] * 5,
        out_specs=pl.BlockSpec(memory_space=pltpu.VMEM),
        scratch_shapes=[
            pltpu.VMEM((n_exp_local, d, h), jnp.bfloat16),
            pltpu.VMEM((SLOTS, n_exp_local, d, h), jnp.bfloat16),
            pltpu.VMEM((SLOTS, n_exp_local, d, h), jnp.bfloat16),
            pltpu.SemaphoreType.DMA((SLOTS, C)),
            pltpu.SemaphoreType.DMA((SLOTS, C)),
            pltpu.SemaphoreType.DMA((SLOTS, C)),
            pltpu.SemaphoreType.DMA((SLOTS, C)),
        ],
        compiler_params=pltpu.CompilerParams(collective_id=0),
    )(x, router_W, route_idx, expert_W, shared_W)
